# baseline (speedup 1.0000x reference)
import functools

import jax
import jax.numpy as jnp
from jax import lax
from jax.experimental import pallas as pl
from jax.experimental.pallas import tpu as pltpu
from jax.experimental.pallas import tpu_sc as plsc

N = 10000
E = 320000
D = 128
NPAD = 10240
NSC = 2
NTILE = 16
K = 128
CPT = 160
BC = 40
NBUF = 2
EPAD = NTILE * K * CPT
ROWS_PT = NPAD // NTILE
GK = 40
GROWS = NPAD // (NSC * NTILE)
GC = GROWS // GK
G2C = 2 * GC

_mesh = plsc.VectorSubcoreMesh(core_axis_name="c", subcore_axis_name="s")


@functools.partial(
    pl.kernel,
    out_type=jax.ShapeDtypeStruct((NSC * NPAD, D), jnp.float32),
    mesh=_mesh,
    scratch_types=[
        pltpu.VMEM((2 * BC, K), jnp.int32),
        pltpu.VMEM((BC, K), jnp.float32),
        [pltpu.VMEM((K, D), jnp.float32)] * NBUF,
        pltpu.VMEM_SHARED((NPAD, D), jnp.float32),
        [pltpu.SemaphoreType.DMA] * NBUF,
    ],
)
def _spmm2(edge_hbm, val_hbm, x_hbm, out_hbm, edge_v, val_v, bufs, acc, sems):
    cid = lax.axis_index("c")
    sid = lax.axis_index("s")
    gbuf = bufs[0]

    def _zrow(r, _):
        for f in range(D // 16):
            gbuf[r, pl.ds(f * 16, 16)] = jnp.zeros((16,), jnp.float32)
        return 0
    lax.fori_loop(0, K, _zrow, 0)

    def _zcopy(j, _):
        pltpu.sync_copy(gbuf, acc.at[pl.ds(sid * ROWS_PT + j * K, K)])
        return 0
    lax.fori_loop(0, ROWS_PT // K, _zcopy, 0)
    plsc.subcore_barrier()

    rbase = (cid * NTILE + sid) * CPT
    dnums = lax.GatherDimensionNumbers(
        offset_dims=(), collapsed_slice_dims=(0,), start_index_map=(0,))
    pib = lax.GatherScatterMode.PROMISE_IN_BOUNDS

    def _scale(buf, c):
        def _grp(g, _):
            vals16 = val_v[c, pl.ds(g * 16, 16)]
            for j in range(16):
                e = g * 16 + j
                v = lax.gather(vals16, jnp.full((16, 1), j, jnp.int32),
                               dnums, (1,), mode=pib)
                for f in range(D // 16):
                    buf[e, pl.ds(f * 16, 16)] = buf[e, pl.ds(f * 16, 16)] * v
            return 0
        lax.fori_loop(0, K // 16, _grp, 0)

    def _block(b, _):
        pltpu.sync_copy(edge_hbm.at[pl.ds(2 * (rbase + b * BC), 2 * BC)],
                        edge_v)
        pltpu.sync_copy(val_hbm.at[pl.ds(rbase + b * BC, BC)], val_v)
        for q in range(NBUF - 1):
            pltpu.async_copy(x_hbm.at[edge_v.at[2 * q]], bufs[q], sems[q])

        @pl.loop(0, BC, step=NBUF)
        def _quad(i):
            for q in range(NBUF):
                c = i + q
                pltpu.make_async_copy(
                    x_hbm.at[edge_v.at[2 * c]], bufs[q], sems[q]).wait()
                nq = (q + NBUF - 1) % NBUF

                @pl.when(c + NBUF - 1 < BC)
                def _():
                    pltpu.async_copy(
                        x_hbm.at[edge_v.at[2 * (c + NBUF - 1)]],
                        bufs[nq], sems[nq])

                _scale(bufs[q], c)
                pltpu.sync_copy(bufs[q], acc.at[edge_v.at[2 * c + 1]],
                                add=True)
        return 0
    lax.fori_loop(0, CPT // BC, _block, 0)

    plsc.subcore_barrier()
    pltpu.sync_copy(acc.at[pl.ds(sid * ROWS_PT, ROWS_PT)],
                    out_hbm.at[pl.ds(cid * NPAD + sid * ROWS_PT, ROWS_PT)])


@functools.partial(
    pl.kernel,
    out_type=jax.ShapeDtypeStruct((NSC * NPAD, D), jnp.float32),
    mesh=_mesh,
    scratch_types=[
        pltpu.VMEM((G2C, GK), jnp.int32),
        [pltpu.VMEM((GK, D), jnp.float32)] * 4,
        [pltpu.SemaphoreType.DMA] * 4,
    ],
)
def _gather2(t_hbm, lbl_hbm, out_hbm, idx_v, bufs, sems):
    cid = lax.axis_index("c")
    sid = lax.axis_index("s")
    wid = cid * NTILE + sid
    pltpu.sync_copy(lbl_hbm.at[pl.ds(wid * G2C, G2C)], idx_v)

    for q in range(4 - 1):
        pltpu.async_copy(t_hbm.at[idx_v.at[q]], bufs[q], sems[q])

    @pl.loop(0, G2C, step=4)
    def _jbody(j):
        for q in range(4):
            jj = j + q
            nq = (q + 3) % 4
            pltpu.make_async_copy(
                t_hbm.at[idx_v.at[jj]], bufs[q], sems[q]).wait()

            @pl.when(jj + 3 < G2C)
            def _():
                pltpu.async_copy(
                    t_hbm.at[idx_v.at[jj + 3]], bufs[nq], sems[nq])

            pltpu.sync_copy(
                bufs[q], out_hbm.at[pl.ds(wid * 2 * GROWS + jj * GK, GK)])


def _dense_body(agg_ref, x_ref, wsp_ref, wtp_ref,
                gsp_ref, bsp_ref, gtp_ref, btp_ref, o_ref):
    def one(agg, x, w, g, b, lo, hi):
        h = jnp.dot(agg, w, preferred_element_type=jnp.float32)
        mu = jnp.mean(h, axis=0, keepdims=True)
        var = jnp.mean((h - mu) ** 2, axis=0, keepdims=True)
        hn = (h - mu) * lax.rsqrt(var + 1e-3) * g + b
        o_ref[lo:hi] = jnp.maximum(hn + x, 0.0)

    one(agg_ref[0:N], x_ref[0:N], wsp_ref[:], gsp_ref[:], bsp_ref[:], 0, N)
    one(agg_ref[NPAD:NPAD + N], x_ref[N:2 * N], wtp_ref[:], gtp_ref[:],
        btp_ref[:], N, 2 * N)


_dense = pl.pallas_call(
    _dense_body,
    out_shape=jax.ShapeDtypeStruct((2 * N, D), jnp.float32),
)


def _head_body(g_ref, spW1, spb1, spW2, spb2, tpW1, tpb1, tpW2, tpb2,
               W1a, W1b, b1, W2, b2, W3, b3, o_ref):
    def mm(a, w):
        return jnp.dot(a, w, preferred_element_type=jnp.float32)

    sg = g_ref[0:N]
    tg = g_ref[NPAD:NPAD + N]
    sp = jnp.maximum(mm(sg, spW1[:]) + spb1[:], 0.0)
    sp = mm(sp, spW2[:]) + spb2[:]
    tp = jnp.maximum(mm(tg, tpW1[:]) + tpb1[:], 0.0)
    tp = mm(tp, tpW2[:]) + tpb2[:]
    z = jnp.maximum(mm(sp, W1a[:]) + mm(tp, W1b[:]) + b1[:], 0.0)
    z = jnp.maximum(mm(z, W2[:]) + b2[:], 0.0)
    o_ref[:] = mm(z, W3[:]) + b3[:]


_head = pl.pallas_call(
    _head_body,
    out_shape=jax.ShapeDtypeStruct((N, D), jnp.float32),
)


def kernel(sp_adj_idx, tp_adj_idx, sp_adj_val, tp_adj_val,
           sp_feat, tp_feat, label_idx, params):
    p = params
    i32 = jnp.int32
    rows = NSC * EPAD // K
    epad = EPAD - E

    zi = jnp.zeros((epad,), i32)
    src2 = jnp.concatenate(
        [sp_adj_idx[:, 1].astype(i32), zi,
         tp_adj_idx[:, 1].astype(i32) + N, zi]).reshape(rows, K)
    dst2 = jnp.concatenate(
        [sp_adj_idx[:, 0].astype(i32), zi,
         tp_adj_idx[:, 0].astype(i32), zi]).reshape(rows, K)
    val2 = jnp.concatenate(
        [sp_adj_val, jnp.zeros((epad,), jnp.float32),
         tp_adj_val, jnp.zeros((epad,), jnp.float32)]).reshape(rows, K)
    edge2 = jnp.stack([src2, dst2], axis=1).reshape(2 * rows, K)
    lblp = jnp.concatenate(
        [label_idx.astype(i32), jnp.zeros((NPAD - N,), i32)])
    lbl2 = jnp.concatenate([lblp, lblp + N]).reshape(2 * NPAD // GK, GK)

    r1 = lambda v: v.reshape(1, D)
    gsp1, bsp1 = r1(p["g_sp1"]), r1(p["b_sp1"])
    gsp2, bsp2 = r1(p["g_sp2"]), r1(p["b_sp2"])
    gtp1, btp1 = r1(p["g_tp1"]), r1(p["b_tp1"])
    gtp2, btp2 = r1(p["g_tp2"]), r1(p["b_tp2"])
    W1a, W1b = p["c_W1"][0:D], p["c_W1"][D:2 * D]
    W3 = jnp.pad(p["c_W3"], ((0, 0), (0, D - 2)))
    b3 = jnp.pad(p["c_b3"], (0, D - 2)).reshape(1, D)

    xcat = jnp.concatenate([sp_feat, tp_feat], axis=0)
    for layer in (1, 2):
        parts = _spmm2(edge2, val2, xcat)
        if layer == 1:
            xcat = _dense(parts, xcat, p["W_sp1"], p["W_tp1"],
                          gsp1, bsp1, gtp1, btp1)
        else:
            xcat = _dense(parts, xcat, p["W_sp2"], p["W_tp2"],
                          gsp2, bsp2, gtp2, btp2)

    g = _gather2(xcat, lbl2)
    z = _head(g, p["spc_W1"], r1(p["spc_b1"]), p["spc_W2"], r1(p["spc_b2"]),
              p["tpc_W1"], r1(p["tpc_b1"]), p["tpc_W2"], r1(p["tpc_b2"]),
              W1a, W1b, p["c_b1"].reshape(1, 2 * D), p["c_W2"],
              r1(p["c_b2"]), W3, b3)
    return z[:, :2]
